# baseline (speedup 1.0000x reference)
import functools

import jax
import jax.numpy as jnp
from jax import lax
from jax.experimental import pallas as pl
from jax.experimental.pallas import tpu as pltpu
from jax.experimental.pallas import tpu_sc as plsc

N = 10000
FEATS = 128
HID = 64
CLASSES = 128

NC = 2
NS = 16
NW = NC * NS
CH = 256

NP = 10240
RPT = NP // NS

_mesh = plsc.VectorSubcoreMesh(core_axis_name="c", subcore_axis_name="s")


def _deg_body(src_hbm, dst_hbm, ones_hbm, zd_hbm, dego_hbm, degi_hbm,
              src_v, dst_v, ones_v, zbuf_v, dego_sh, degi_sh):
    nchunk = src_v.shape[0]
    c = lax.axis_index("c")
    s = lax.axis_index("s")
    wid = c * NS + s
    pltpu.sync_copy(ones_hbm, ones_v)
    pltpu.sync_copy(zd_hbm, zbuf_v)
    pltpu.sync_copy(zbuf_v, dego_sh.at[pl.ds(s * RPT, RPT)])
    pltpu.sync_copy(zbuf_v, degi_sh.at[pl.ds(s * RPT, RPT)])
    pltpu.sync_copy(src_hbm.at[wid], src_v)
    pltpu.sync_copy(dst_hbm.at[wid], dst_v)
    plsc.subcore_barrier()

    def body(j, carry):
        pltpu.sync_copy(ones_v, dego_sh.at[src_v.at[j]], add=True)
        pltpu.sync_copy(ones_v, degi_sh.at[dst_v.at[j]], add=True)
        return carry

    lax.fori_loop(0, nchunk, body, 0)
    plsc.subcore_barrier()
    pltpu.sync_copy(dego_sh.at[pl.ds(s * RPT, RPT)], zbuf_v)
    pltpu.sync_copy(zbuf_v, dego_hbm.at[c, pl.ds(s * RPT, RPT)])
    pltpu.sync_copy(degi_sh.at[pl.ds(s * RPT, RPT)], zbuf_v)
    pltpu.sync_copy(zbuf_v, degi_hbm.at[c, pl.ds(s * RPT, RPT)])


def _agg_body(hs_hbm, src_hbm, dst_hbm, z_hbm, out_hbm,
              src_v, dst_v, rows0_v, zbuf_v, acc_sh, semg0):
    nchunk = src_v.shape[0]
    c = lax.axis_index("c")
    s = lax.axis_index("s")
    wid = c * NS + s
    pltpu.sync_copy(z_hbm, zbuf_v)
    pltpu.sync_copy(zbuf_v, acc_sh.at[pl.ds(s * RPT, RPT)])
    pltpu.sync_copy(src_hbm.at[wid], src_v)
    pltpu.sync_copy(dst_hbm.at[wid], dst_v)
    plsc.subcore_barrier()

    def body(j, carry):
        pltpu.async_copy(hs_hbm.at[src_v.at[j]], rows0_v, semg0).wait()
        pltpu.sync_copy(rows0_v, acc_sh.at[dst_v.at[j]], add=True)
        return carry

    lax.fori_loop(0, nchunk, body, 0)
    plsc.subcore_barrier()
    pltpu.sync_copy(acc_sh.at[pl.ds(s * RPT, RPT)], zbuf_v)
    pltpu.sync_copy(zbuf_v, out_hbm.at[c, pl.ds(s * RPT, RPT)])


def _deg_call(src_p, dst_p, ones8, z8):
    nchunk = src_p.shape[1]
    f = pl.kernel(
        _deg_body,
        out_type=(jax.ShapeDtypeStruct((NC, NP, 8), jnp.float32),
                  jax.ShapeDtypeStruct((NC, NP, 8), jnp.float32)),
        mesh=_mesh,
        scratch_types=[
            pltpu.VMEM((nchunk, CH), jnp.int32),
            pltpu.VMEM((nchunk, CH), jnp.int32),
            pltpu.VMEM((CH, 8), jnp.float32),
            pltpu.VMEM((RPT, 8), jnp.float32),
            pltpu.VMEM_SHARED((NP, 8), jnp.float32),
            pltpu.VMEM_SHARED((NP, 8), jnp.float32),
        ],
        compiler_params=pltpu.CompilerParams(use_tc_tiling_on_sc=False),
    )
    return f(src_p, dst_p, ones8, z8)


def _agg_call(hs, src_p, dst_p, z64):
    nchunk = src_p.shape[1]
    f = pl.kernel(
        _agg_body,
        out_type=jax.ShapeDtypeStruct((NC, NP, HID), jnp.float32),
        mesh=_mesh,
        scratch_types=[
            pltpu.VMEM((nchunk, CH), jnp.int32),
            pltpu.VMEM((nchunk, CH), jnp.int32),
            pltpu.VMEM((CH, HID), jnp.float32),
            pltpu.VMEM((RPT, HID), jnp.float32),
            pltpu.VMEM_SHARED((NP, HID), jnp.float32),
            pltpu.SemaphoreType.DMA,
        ],
        compiler_params=pltpu.CompilerParams(use_tc_tiling_on_sc=False),
    )
    return f(hs, src_p, dst_p, z64)


_ROWS = 1024


def _l1_body(x_ref, w_ref, b_ref, dego_ref, h_ref, hs_ref):
    z = jnp.dot(x_ref[...], w_ref[...], preferred_element_type=jnp.float32)
    h = jnp.maximum(z + b_ref[...][None, :], 0.0)
    dsum = dego_ref[0, :, 0:1] + dego_ref[1, :, 0:1]
    dout = lax.rsqrt(jnp.maximum(dsum, 1.0))
    h_ref[...] = h
    hs_ref[...] = h * dout


def _mid_body(h_ref, p_ref, degi_ref, dego_ref, wt_ref, wb_ref, b_ref,
              h2_ref, hs2_ref):
    dsum_i = degi_ref[0, :, 0:1] + degi_ref[1, :, 0:1]
    din = lax.rsqrt(jnp.maximum(dsum_i, 1.0))
    g = (p_ref[0] + p_ref[1]) * din
    z = (jnp.dot(h_ref[...], wt_ref[...], preferred_element_type=jnp.float32)
         + jnp.dot(g, wb_ref[...], preferred_element_type=jnp.float32))
    h2 = jnp.maximum(z + b_ref[...][None, :], 0.0)
    dsum_o = dego_ref[0, :, 0:1] + dego_ref[1, :, 0:1]
    dout = lax.rsqrt(jnp.maximum(dsum_o, 1.0))
    h2_ref[...] = h2
    hs2_ref[...] = h2 * dout


def _fin_body(h_ref, p_ref, degi_ref, wt_ref, wb_ref, b_ref, o_ref):
    dsum_i = degi_ref[0, :, 0:1] + degi_ref[1, :, 0:1]
    din = lax.rsqrt(jnp.maximum(dsum_i, 1.0))
    g = (p_ref[0] + p_ref[1]) * din
    z = (jnp.dot(h_ref[...], wt_ref[...], preferred_element_type=jnp.float32)
         + jnp.dot(g, wb_ref[...], preferred_element_type=jnp.float32))
    o_ref[...] = jnp.maximum(z + b_ref[...][None, :], 0.0)


def _row_spec(width):
    return pl.BlockSpec((_ROWS, width), lambda i: (i, 0))


def _pair_spec(width):
    return pl.BlockSpec((NC, _ROWS, width), lambda i: (0, i, 0))


def _full_spec(shape):
    nd = len(shape)
    return pl.BlockSpec(shape, lambda i: (0,) * nd)


def _l1_call(x_pad, W1, b1, dego):
    grid = NP // _ROWS
    return pl.pallas_call(
        _l1_body,
        grid=(grid,),
        in_specs=[_row_spec(FEATS), _full_spec(W1.shape), _full_spec(b1.shape),
                  _pair_spec(8)],
        out_specs=[_row_spec(HID), _row_spec(HID)],
        out_shape=[jax.ShapeDtypeStruct((NP, HID), jnp.float32),
                   jax.ShapeDtypeStruct((NP, HID), jnp.float32)],
        compiler_params=pltpu.CompilerParams(
            dimension_semantics=("parallel",)),
    )(x_pad, W1, b1, dego)


def _mid_call(h, p, degi, dego, Wt, Wb, b):
    grid = NP // _ROWS
    return pl.pallas_call(
        _mid_body,
        grid=(grid,),
        in_specs=[_row_spec(HID), _pair_spec(HID), _pair_spec(8),
                  _pair_spec(8), _full_spec(Wt.shape), _full_spec(Wb.shape),
                  _full_spec(b.shape)],
        out_specs=[_row_spec(HID), _row_spec(HID)],
        out_shape=[jax.ShapeDtypeStruct((NP, HID), jnp.float32),
                   jax.ShapeDtypeStruct((NP, HID), jnp.float32)],
        compiler_params=pltpu.CompilerParams(
            dimension_semantics=("parallel",)),
    )(h, p, degi, dego, Wt, Wb, b)


def _fin_call(h, p, degi, Wt, Wb, b):
    grid = NP // _ROWS
    return pl.pallas_call(
        _fin_body,
        grid=(grid,),
        in_specs=[_row_spec(HID), _pair_spec(HID), _pair_spec(8),
                  _full_spec(Wt.shape), _full_spec(Wb.shape),
                  _full_spec(b.shape)],
        out_specs=_row_spec(CLASSES),
        out_shape=jax.ShapeDtypeStruct((NP, CLASSES), jnp.float32),
        compiler_params=pltpu.CompilerParams(
            dimension_semantics=("parallel",)),
    )(h, p, degi, Wt, Wb, b)


def kernel(x, edges, W1, b1, W2, b2, W3, b3, W4, b4):
    E = edges.shape[1]
    ep = ((E + NW * CH - 1) // (NW * CH)) * (NW * CH)
    src = edges[0].astype(jnp.int32)
    dst = edges[1].astype(jnp.int32)
    if ep > E:
        padv = jnp.full((ep - E,), NP - 1, jnp.int32)
        src = jnp.concatenate([src, padv])
        dst = jnp.concatenate([dst, padv])
    src_p = src.reshape(NW, ep // (NW * CH), CH)
    dst_p = dst.reshape(NW, ep // (NW * CH), CH)
    x_pad = jnp.zeros((NP, FEATS), jnp.float32).at[:N].set(x)
    ones8 = jnp.ones((CH, 8), jnp.float32)
    z8 = jnp.zeros((RPT, 8), jnp.float32)
    z64 = jnp.zeros((RPT, HID), jnp.float32)

    dego, degi = _deg_call(src_p, dst_p, ones8, z8)
    h, hs = _l1_call(x_pad, W1, b1, dego)
    p = _agg_call(hs, src_p, dst_p, z64)
    h, hs = _mid_call(h, p, degi, dego, W2[:HID], W2[HID:], b2)
    p = _agg_call(hs, src_p, dst_p, z64)
    h, hs = _mid_call(h, p, degi, dego, W3[:HID], W3[HID:], b3)
    p = _agg_call(hs, src_p, dst_p, z64)
    out = _fin_call(h, p, degi, W4[:HID], W4[HID:], b4)
    return out[:N]


# speedup vs baseline: 1.2886x; 1.2886x over previous
import functools

import jax
import jax.numpy as jnp
from jax import lax
from jax.experimental import pallas as pl
from jax.experimental.pallas import tpu as pltpu
from jax.experimental.pallas import tpu_sc as plsc

N = 10000
FEATS = 128
HID = 64
CLASSES = 128

NC = 2
NS = 16
NW = NC * NS
CH = 128

NP = 10240
RPT = NP // NS

_mesh = plsc.VectorSubcoreMesh(core_axis_name="c", subcore_axis_name="s")


def _deg_body(src_hbm, dst_hbm, ones_hbm, zd_hbm, dego_hbm, degi_hbm,
              src_v, dst_v, ones_v, zbuf_v, dego_sh, degi_sh):
    nchunk = src_v.shape[0]
    c = lax.axis_index("c")
    s = lax.axis_index("s")
    wid = c * NS + s
    pltpu.sync_copy(ones_hbm, ones_v)
    pltpu.sync_copy(zd_hbm, zbuf_v)
    pltpu.sync_copy(zbuf_v, dego_sh.at[pl.ds(s * RPT, RPT)])
    pltpu.sync_copy(zbuf_v, degi_sh.at[pl.ds(s * RPT, RPT)])
    pltpu.sync_copy(src_hbm.at[wid], src_v)
    pltpu.sync_copy(dst_hbm.at[wid], dst_v)
    plsc.subcore_barrier()

    def body(j, carry):
        pltpu.sync_copy(ones_v, dego_sh.at[src_v.at[j]], add=True)
        pltpu.sync_copy(ones_v, degi_sh.at[dst_v.at[j]], add=True)
        return carry

    lax.fori_loop(0, nchunk, body, 0)
    plsc.subcore_barrier()
    pltpu.sync_copy(dego_sh.at[pl.ds(s * RPT, RPT)], zbuf_v)
    pltpu.sync_copy(zbuf_v, dego_hbm.at[c, pl.ds(s * RPT, RPT)])
    pltpu.sync_copy(degi_sh.at[pl.ds(s * RPT, RPT)], zbuf_v)
    pltpu.sync_copy(zbuf_v, degi_hbm.at[c, pl.ds(s * RPT, RPT)])


def _agg_body(hs_hbm, src_hbm, dst_hbm, z_hbm, out_hbm,
              src_v, dst_v, r0_v, zbuf_v, acc_sh, semg):
    nchunk = src_v.shape[0]
    c = lax.axis_index("c")
    s = lax.axis_index("s")
    wid = c * NS + s
    q = RPT // 4
    pltpu.sync_copy(z_hbm, zbuf_v)
    for t in range(4):
        pltpu.sync_copy(zbuf_v, acc_sh.at[pl.ds(s * RPT + t * q, q)])
    pltpu.sync_copy(src_hbm.at[wid], src_v)
    pltpu.sync_copy(dst_hbm.at[wid], dst_v)
    plsc.subcore_barrier()

    def body(j, carry):
        pltpu.async_copy(hs_hbm.at[src_v.at[j]], r0_v, semg).wait()
        pltpu.sync_copy(r0_v, acc_sh.at[dst_v.at[j]], add=True)
        return carry

    lax.fori_loop(0, nchunk, body, 0)
    plsc.subcore_barrier()
    for t in range(4):
        pltpu.sync_copy(acc_sh.at[pl.ds(s * RPT + t * q, q)], zbuf_v)
        pltpu.sync_copy(zbuf_v, out_hbm.at[c, pl.ds(s * RPT + t * q, q)])


def _deg_call(src_p, dst_p, ones8, z8):
    nchunk = src_p.shape[1]
    f = pl.kernel(
        _deg_body,
        out_type=(jax.ShapeDtypeStruct((NC, NP, 8), jnp.float32),
                  jax.ShapeDtypeStruct((NC, NP, 8), jnp.float32)),
        mesh=_mesh,
        scratch_types=[
            pltpu.VMEM((nchunk, CH), jnp.int32),
            pltpu.VMEM((nchunk, CH), jnp.int32),
            pltpu.VMEM((CH, 8), jnp.float32),
            pltpu.VMEM((RPT, 8), jnp.float32),
            pltpu.VMEM_SHARED((NP, 8), jnp.float32),
            pltpu.VMEM_SHARED((NP, 8), jnp.float32),
        ],
        compiler_params=pltpu.CompilerParams(use_tc_tiling_on_sc=False),
    )
    return f(src_p, dst_p, ones8, z8)


def _agg_call(hs, src_p, dst_p, z64):
    nchunk = src_p.shape[1]
    f = pl.kernel(
        _agg_body,
        out_type=jax.ShapeDtypeStruct((NC, NP, HID), jnp.float32),
        mesh=_mesh,
        scratch_types=[
            pltpu.VMEM((nchunk, CH), jnp.int32),
            pltpu.VMEM((nchunk, CH), jnp.int32),
            pltpu.VMEM((CH, HID), jnp.float32),
            pltpu.VMEM((RPT // 4, HID), jnp.float32),
            pltpu.VMEM_SHARED((NP, HID), jnp.float32),
            pltpu.SemaphoreType.DMA,
        ],
        compiler_params=pltpu.CompilerParams(use_tc_tiling_on_sc=False),
    )
    return f(hs, src_p, dst_p, z64)


_ROWS = 1024


def _l1_body(x_ref, w_ref, b_ref, dego_ref, h_ref, hs_ref):
    z = jnp.dot(x_ref[...], w_ref[...], preferred_element_type=jnp.float32)
    h = jnp.maximum(z + b_ref[...][None, :], 0.0)
    dsum = dego_ref[0, :, 0:1] + dego_ref[1, :, 0:1]
    dout = lax.rsqrt(jnp.maximum(dsum, 1.0))
    h_ref[...] = h
    hs_ref[...] = h * dout


def _mid_body(h_ref, p_ref, degi_ref, dego_ref, wt_ref, wb_ref, b_ref,
              h2_ref, hs2_ref):
    dsum_i = degi_ref[0, :, 0:1] + degi_ref[1, :, 0:1]
    din = lax.rsqrt(jnp.maximum(dsum_i, 1.0))
    g = (p_ref[0] + p_ref[1]) * din
    z = (jnp.dot(h_ref[...], wt_ref[...], preferred_element_type=jnp.float32)
         + jnp.dot(g, wb_ref[...], preferred_element_type=jnp.float32))
    h2 = jnp.maximum(z + b_ref[...][None, :], 0.0)
    dsum_o = dego_ref[0, :, 0:1] + dego_ref[1, :, 0:1]
    dout = lax.rsqrt(jnp.maximum(dsum_o, 1.0))
    h2_ref[...] = h2
    hs2_ref[...] = h2 * dout


def _fin_body(h_ref, p_ref, degi_ref, wt_ref, wb_ref, b_ref, o_ref):
    dsum_i = degi_ref[0, :, 0:1] + degi_ref[1, :, 0:1]
    din = lax.rsqrt(jnp.maximum(dsum_i, 1.0))
    g = (p_ref[0] + p_ref[1]) * din
    z = (jnp.dot(h_ref[...], wt_ref[...], preferred_element_type=jnp.float32)
         + jnp.dot(g, wb_ref[...], preferred_element_type=jnp.float32))
    o_ref[...] = jnp.maximum(z + b_ref[...][None, :], 0.0)


def _row_spec(width):
    return pl.BlockSpec((_ROWS, width), lambda i: (i, 0))


def _pair_spec(width):
    return pl.BlockSpec((NC, _ROWS, width), lambda i: (0, i, 0))


def _full_spec(shape):
    nd = len(shape)
    return pl.BlockSpec(shape, lambda i: (0,) * nd)


def _l1_call(x_pad, W1, b1, dego):
    grid = NP // _ROWS
    return pl.pallas_call(
        _l1_body,
        grid=(grid,),
        in_specs=[_row_spec(FEATS), _full_spec(W1.shape), _full_spec(b1.shape),
                  _pair_spec(8)],
        out_specs=[_row_spec(HID), _row_spec(HID)],
        out_shape=[jax.ShapeDtypeStruct((NP, HID), jnp.float32),
                   jax.ShapeDtypeStruct((NP, HID), jnp.float32)],
        compiler_params=pltpu.CompilerParams(
            dimension_semantics=("parallel",)),
    )(x_pad, W1, b1, dego)


def _mid_call(h, p, degi, dego, Wt, Wb, b):
    grid = NP // _ROWS
    return pl.pallas_call(
        _mid_body,
        grid=(grid,),
        in_specs=[_row_spec(HID), _pair_spec(HID), _pair_spec(8),
                  _pair_spec(8), _full_spec(Wt.shape), _full_spec(Wb.shape),
                  _full_spec(b.shape)],
        out_specs=[_row_spec(HID), _row_spec(HID)],
        out_shape=[jax.ShapeDtypeStruct((NP, HID), jnp.float32),
                   jax.ShapeDtypeStruct((NP, HID), jnp.float32)],
        compiler_params=pltpu.CompilerParams(
            dimension_semantics=("parallel",)),
    )(h, p, degi, dego, Wt, Wb, b)


def _fin_call(h, p, degi, Wt, Wb, b):
    grid = NP // _ROWS
    return pl.pallas_call(
        _fin_body,
        grid=(grid,),
        in_specs=[_row_spec(HID), _pair_spec(HID), _pair_spec(8),
                  _full_spec(Wt.shape), _full_spec(Wb.shape),
                  _full_spec(b.shape)],
        out_specs=_row_spec(CLASSES),
        out_shape=jax.ShapeDtypeStruct((NP, CLASSES), jnp.float32),
        compiler_params=pltpu.CompilerParams(
            dimension_semantics=("parallel",)),
    )(h, p, degi, Wt, Wb, b)


def kernel(x, edges, W1, b1, W2, b2, W3, b3, W4, b4):
    E = edges.shape[1]
    ep = ((E + NW * CH - 1) // (NW * CH)) * (NW * CH)
    src = edges[0].astype(jnp.int32)
    dst = edges[1].astype(jnp.int32)
    if ep > E:
        padv = jnp.full((ep - E,), NP - 1, jnp.int32)
        src = jnp.concatenate([src, padv])
        dst = jnp.concatenate([dst, padv])
    src_p = src.reshape(NW, ep // (NW * CH), CH)
    dst_p = dst.reshape(NW, ep // (NW * CH), CH)
    x_pad = jnp.zeros((NP, FEATS), jnp.float32).at[:N].set(x)
    ones8 = jnp.ones((CH, 8), jnp.float32)
    z8 = jnp.zeros((RPT, 8), jnp.float32)
    z64 = jnp.zeros((RPT // 4, HID), jnp.float32)

    dego, degi = _deg_call(src_p, dst_p, ones8, z8)
    h, hs = _l1_call(x_pad, W1, b1, dego)
    p = _agg_call(hs, src_p, dst_p, z64)
    h, hs = _mid_call(h, p, degi, dego, W2[:HID], W2[HID:], b2)
    p = _agg_call(hs, src_p, dst_p, z64)
    h, hs = _mid_call(h, p, degi, dego, W3[:HID], W3[HID:], b3)
    p = _agg_call(hs, src_p, dst_p, z64)
    out = _fin_call(h, p, degi, W4[:HID], W4[HID:], b4)
    return out[:N]


# speedup vs baseline: 1.3100x; 1.0166x over previous
import functools

import jax
import jax.numpy as jnp
from jax import lax
from jax.experimental import pallas as pl
from jax.experimental.pallas import tpu as pltpu
from jax.experimental.pallas import tpu_sc as plsc

N = 10000
FEATS = 128
HID = 64
CLASSES = 128

NC = 2
NS = 16
NW = NC * NS
CH = 128

NP = 10240
RPT = NP // NS

_mesh = plsc.VectorSubcoreMesh(core_axis_name="c", subcore_axis_name="s")


def _deg_body(src_hbm, dst_hbm, ones_hbm, zd_hbm, dego_hbm, degi_hbm,
              src_v, dst_v, ones_v, zbuf_v, dego_sh, degi_sh):
    nchunk = src_v.shape[0]
    c = lax.axis_index("c")
    s = lax.axis_index("s")
    wid = c * NS + s
    pltpu.sync_copy(ones_hbm, ones_v)
    pltpu.sync_copy(zd_hbm, zbuf_v)
    pltpu.sync_copy(zbuf_v, dego_sh.at[pl.ds(s * RPT, RPT)])
    pltpu.sync_copy(zbuf_v, degi_sh.at[pl.ds(s * RPT, RPT)])
    pltpu.sync_copy(src_hbm.at[wid], src_v)
    pltpu.sync_copy(dst_hbm.at[wid], dst_v)
    plsc.subcore_barrier()

    def body(j, carry):
        pltpu.sync_copy(ones_v, dego_sh.at[src_v.at[j]], add=True)
        pltpu.sync_copy(ones_v, degi_sh.at[dst_v.at[j]], add=True)
        return carry

    lax.fori_loop(0, nchunk, body, 0)
    plsc.subcore_barrier()
    pltpu.sync_copy(dego_sh.at[pl.ds(s * RPT, RPT)], zbuf_v)
    pltpu.sync_copy(zbuf_v, dego_hbm.at[c, pl.ds(s * RPT, RPT)])
    pltpu.sync_copy(degi_sh.at[pl.ds(s * RPT, RPT)], zbuf_v)
    pltpu.sync_copy(zbuf_v, degi_hbm.at[c, pl.ds(s * RPT, RPT)])


def _agg_body(hs_hbm, src_hbm, dst_hbm, z_hbm, out_hbm,
              src_v, dst_v, r0_v, zbuf_v, acc_sh, semg):
    nchunk = src_v.shape[0]
    c = lax.axis_index("c")
    s = lax.axis_index("s")
    wid = c * NS + s
    q = RPT // 4
    pltpu.sync_copy(z_hbm, zbuf_v)
    for t in range(4):
        pltpu.sync_copy(zbuf_v, acc_sh.at[pl.ds(s * RPT + t * q, q)])
    pltpu.sync_copy(src_hbm.at[wid], src_v)
    pltpu.sync_copy(dst_hbm.at[wid], dst_v)
    plsc.subcore_barrier()

    def body(j, carry):
        pltpu.async_copy(hs_hbm.at[src_v.at[j]], r0_v, semg).wait()
        pltpu.sync_copy(r0_v, acc_sh.at[dst_v.at[j]], add=True)
        return carry

    lax.fori_loop(0, nchunk, body, 0)
    plsc.subcore_barrier()
    for t in range(4):
        pltpu.sync_copy(acc_sh.at[pl.ds(s * RPT + t * q, q)], zbuf_v)
        pltpu.sync_copy(zbuf_v, out_hbm.at[c, pl.ds(s * RPT + t * q, q)])


def _deg_call(src_p, dst_p, ones8, z8):
    nchunk = src_p.shape[1]
    f = pl.kernel(
        _deg_body,
        out_type=(jax.ShapeDtypeStruct((NC, NP, 8), jnp.float32),
                  jax.ShapeDtypeStruct((NC, NP, 8), jnp.float32)),
        mesh=_mesh,
        scratch_types=[
            pltpu.VMEM((nchunk, CH), jnp.int32),
            pltpu.VMEM((nchunk, CH), jnp.int32),
            pltpu.VMEM((CH, 8), jnp.float32),
            pltpu.VMEM((RPT, 8), jnp.float32),
            pltpu.VMEM_SHARED((NP, 8), jnp.float32),
            pltpu.VMEM_SHARED((NP, 8), jnp.float32),
        ],
        compiler_params=pltpu.CompilerParams(use_tc_tiling_on_sc=False),
    )
    return f(src_p, dst_p, ones8, z8)


@functools.lru_cache(maxsize=None)
def _agg_kernel(nchunk):
    return pl.kernel(
        _agg_body,
        out_type=jax.ShapeDtypeStruct((NC, NP, HID), jnp.float32),
        mesh=_mesh,
        scratch_types=[
            pltpu.VMEM((nchunk, CH), jnp.int32),
            pltpu.VMEM((nchunk, CH), jnp.int32),
            pltpu.VMEM((CH, HID), jnp.float32),
            pltpu.VMEM((RPT // 4, HID), jnp.float32),
            pltpu.VMEM_SHARED((NP, HID), jnp.float32),
            pltpu.SemaphoreType.DMA,
        ],
        compiler_params=pltpu.CompilerParams(use_tc_tiling_on_sc=False),
    )


def _agg_call(hs, src_p, dst_p, z64):
    return _agg_kernel(src_p.shape[1])(hs, src_p, dst_p, z64)


_ROWS = 1024


def _l1_body(x_ref, w_ref, b_ref, dego_ref, h_ref, hs_ref):
    z = jnp.dot(x_ref[...], w_ref[...], preferred_element_type=jnp.float32)
    h = jnp.maximum(z + b_ref[...][None, :], 0.0)
    dsum = dego_ref[0, :, 0:1] + dego_ref[1, :, 0:1]
    dout = lax.rsqrt(jnp.maximum(dsum, 1.0))
    h_ref[...] = h
    hs_ref[...] = h * dout


def _mid_body(h_ref, p_ref, degi_ref, dego_ref, wt_ref, wb_ref, b_ref,
              h2_ref, hs2_ref):
    dsum_i = degi_ref[0, :, 0:1] + degi_ref[1, :, 0:1]
    din = lax.rsqrt(jnp.maximum(dsum_i, 1.0))
    g = (p_ref[0] + p_ref[1]) * din
    z = (jnp.dot(h_ref[...], wt_ref[...], preferred_element_type=jnp.float32)
         + jnp.dot(g, wb_ref[...], preferred_element_type=jnp.float32))
    h2 = jnp.maximum(z + b_ref[...][None, :], 0.0)
    dsum_o = dego_ref[0, :, 0:1] + dego_ref[1, :, 0:1]
    dout = lax.rsqrt(jnp.maximum(dsum_o, 1.0))
    h2_ref[...] = h2
    hs2_ref[...] = h2 * dout


def _fin_body(h_ref, p_ref, degi_ref, wt_ref, wb_ref, b_ref, o_ref):
    dsum_i = degi_ref[0, :, 0:1] + degi_ref[1, :, 0:1]
    din = lax.rsqrt(jnp.maximum(dsum_i, 1.0))
    g = (p_ref[0] + p_ref[1]) * din
    z = (jnp.dot(h_ref[...], wt_ref[...], preferred_element_type=jnp.float32)
         + jnp.dot(g, wb_ref[...], preferred_element_type=jnp.float32))
    o_ref[...] = jnp.maximum(z + b_ref[...][None, :], 0.0)


def _row_spec(width):
    return pl.BlockSpec((_ROWS, width), lambda i: (i, 0))


def _pair_spec(width):
    return pl.BlockSpec((NC, _ROWS, width), lambda i: (0, i, 0))


def _full_spec(shape):
    nd = len(shape)
    return pl.BlockSpec(shape, lambda i: (0,) * nd)


def _l1_call(x_pad, W1, b1, dego):
    grid = NP // _ROWS
    return pl.pallas_call(
        _l1_body,
        grid=(grid,),
        in_specs=[_row_spec(FEATS), _full_spec(W1.shape), _full_spec(b1.shape),
                  _pair_spec(8)],
        out_specs=[_row_spec(HID), _row_spec(HID)],
        out_shape=[jax.ShapeDtypeStruct((NP, HID), jnp.float32),
                   jax.ShapeDtypeStruct((NP, HID), jnp.float32)],
        compiler_params=pltpu.CompilerParams(
            dimension_semantics=("parallel",)),
    )(x_pad, W1, b1, dego)


def _mid_call(h, p, degi, dego, Wt, Wb, b):
    grid = NP // _ROWS
    return pl.pallas_call(
        _mid_body,
        grid=(grid,),
        in_specs=[_row_spec(HID), _pair_spec(HID), _pair_spec(8),
                  _pair_spec(8), _full_spec(Wt.shape), _full_spec(Wb.shape),
                  _full_spec(b.shape)],
        out_specs=[_row_spec(HID), _row_spec(HID)],
        out_shape=[jax.ShapeDtypeStruct((NP, HID), jnp.float32),
                   jax.ShapeDtypeStruct((NP, HID), jnp.float32)],
        compiler_params=pltpu.CompilerParams(
            dimension_semantics=("parallel",)),
    )(h, p, degi, dego, Wt, Wb, b)


def _fin_call(h, p, degi, Wt, Wb, b):
    grid = NP // _ROWS
    return pl.pallas_call(
        _fin_body,
        grid=(grid,),
        in_specs=[_row_spec(HID), _pair_spec(HID), _pair_spec(8),
                  _full_spec(Wt.shape), _full_spec(Wb.shape),
                  _full_spec(b.shape)],
        out_specs=_row_spec(CLASSES),
        out_shape=jax.ShapeDtypeStruct((NP, CLASSES), jnp.float32),
        compiler_params=pltpu.CompilerParams(
            dimension_semantics=("parallel",)),
    )(h, p, degi, Wt, Wb, b)


def kernel(x, edges, W1, b1, W2, b2, W3, b3, W4, b4):
    E = edges.shape[1]
    ep = ((E + NW * CH - 1) // (NW * CH)) * (NW * CH)
    src = edges[0].astype(jnp.int32)
    dst = edges[1].astype(jnp.int32)
    if ep > E:
        padv = jnp.full((ep - E,), NP - 1, jnp.int32)
        src = jnp.concatenate([src, padv])
        dst = jnp.concatenate([dst, padv])
    src_p = src.reshape(NW, ep // (NW * CH), CH)
    dst_p = dst.reshape(NW, ep // (NW * CH), CH)
    x_pad = jnp.zeros((NP, FEATS), jnp.float32).at[:N].set(x)
    ones8 = jnp.ones((CH, 8), jnp.float32)
    z8 = jnp.zeros((RPT, 8), jnp.float32)
    z64 = jnp.zeros((RPT // 4, HID), jnp.float32)

    dego, degi = _deg_call(src_p, dst_p, ones8, z8)
    h, hs = _l1_call(x_pad, W1, b1, dego)
    p = _agg_call(hs, src_p, dst_p, z64)
    h, hs = _mid_call(h, p, degi, dego, W2[:HID], W2[HID:], b2)
    p = _agg_call(hs, src_p, dst_p, z64)
    h, hs = _mid_call(h, p, degi, dego, W3[:HID], W3[HID:], b3)
    p = _agg_call(hs, src_p, dst_p, z64)
    out = _fin_call(h, p, degi, W4[:HID], W4[HID:], b4)
    return out[:N]
